# Initial kernel scaffold; baseline (speedup 1.0000x reference)
import jax
import jax.numpy as jnp
from jax.experimental import pallas as pl


def kernel(x, edge_index, batch, W1, b1, W2, b2, W3, b3, Wo, bo):
    raise NotImplementedError("write your pallas kernel here")


# speedup vs baseline: 14.1633x; 14.1633x over previous
import functools

import jax
import jax.numpy as jnp
from jax import lax
from jax.experimental import pallas as pl
from jax.experimental.pallas import tpu as pltpu
from jax.experimental.pallas import tpu_sc as plsc

N = 10000
E = 320000
D = 128
G = 128
NCLS = 10
NC, NS = 2, 16
NW = NC * NS
EPW = E // NW
CHUNK = 80
NCHUNK = EPW // CHUNK
RPT = 624
TAIL = N - NS * RPT
DEGW = 16
RB = 1000

def _mesh():
    return plsc.VectorSubcoreMesh(core_axis_name="c", subcore_axis_name="s",
                                  num_cores=NC, num_subcores=NS)


def _sc_degree(ones_hbm, dst_w):
    @functools.partial(
        pl.kernel,
        out_type=jax.ShapeDtypeStruct((NC, N, DEGW), jnp.float32),
        mesh=_mesh(),
        scratch_types=[
            pltpu.VMEM((NCHUNK, CHUNK), jnp.int32),
            pltpu.VMEM((CHUNK, DEGW), jnp.float32),
            pltpu.VMEM_SHARED((N, DEGW), jnp.float32),
        ],
    )
    def k(ones_ref, dst_ref, out_ref, idx_v, ones_v, acc):
        cid = lax.axis_index("c")
        sid = lax.axis_index("s")
        w = cid * NS + sid
        pltpu.sync_copy(dst_ref.at[w], idx_v)
        pltpu.sync_copy(ones_ref.at[pl.ds(0, CHUNK)], ones_v)
        rs = pl.ds(sid * RPT, RPT)
        ts = pl.ds(NS * RPT, TAIL)
        pltpu.sync_copy(ones_ref.at[pl.ds(0, RPT)], acc.at[rs])

        @pl.when(sid == NS - 1)
        def _():
            pltpu.sync_copy(ones_ref.at[pl.ds(0, TAIL)], acc.at[ts])

        plsc.subcore_barrier()

        @pl.loop(0, NCHUNK)
        def _(j):
            pltpu.sync_copy(ones_v, acc.at[idx_v.at[j]], add=True)

        plsc.subcore_barrier()
        pltpu.sync_copy(acc.at[rs], out_ref.at[cid, rs])

        @pl.when(sid == NS - 1)
        def _():
            pltpu.sync_copy(acc.at[ts], out_ref.at[cid, ts])

    return k(ones_hbm, dst_w)


def _sc_aggregate(t, src_w, dst_w):
    @functools.partial(
        pl.kernel,
        out_type=jax.ShapeDtypeStruct((NC, N, D), jnp.float32),
        mesh=_mesh(),
        scratch_types=[
            pltpu.VMEM((NCHUNK, CHUNK), jnp.int32),
            pltpu.VMEM((NCHUNK, CHUNK), jnp.int32),
            pltpu.VMEM((CHUNK, D), jnp.float32),
            pltpu.VMEM_SHARED((N, D), jnp.float32),
        ],
    )
    def k(t_ref, src_ref, dst_ref, out_ref, srcv, dstv, rowbuf, acc):
        cid = lax.axis_index("c")
        sid = lax.axis_index("s")
        w = cid * NS + sid
        pltpu.sync_copy(src_ref.at[w], srcv)
        pltpu.sync_copy(dst_ref.at[w], dstv)
        rs = pl.ds(sid * RPT, RPT)
        ts = pl.ds(NS * RPT, TAIL)
        pltpu.sync_copy(t_ref.at[rs], acc.at[rs])

        @pl.when(sid == NS - 1)
        def _():
            pltpu.sync_copy(t_ref.at[ts], acc.at[ts])

        plsc.subcore_barrier()

        @pl.loop(0, NCHUNK)
        def _(j):
            pltpu.sync_copy(t_ref.at[srcv.at[j]], rowbuf)
            pltpu.sync_copy(rowbuf, acc.at[dstv.at[j]], add=True)

        plsc.subcore_barrier()
        pltpu.sync_copy(acc.at[rs], out_ref.at[cid, rs])

        @pl.when(sid == NS - 1)
        def _():
            pltpu.sync_copy(acc.at[ts], out_ref.at[cid, ts])

    return k(t, src_w, dst_w)


def _elu(v):
    return jnp.where(v > 0, v, jnp.exp(jnp.where(v > 0, 0.0, v)) - 1.0)


def _dot(a, b):
    return jnp.dot(a, b, preferred_element_type=jnp.float32,
                   precision=lax.Precision.HIGHEST)


def _tc_layer1(x, w1, cnt):
    def body(x_ref, w_ref, ca_ref, cb_ref, t_ref, dinv_ref):
        deg = ca_ref[0, :, 0:1] + cb_ref[0, :, 0:1] - 1.0
        dinv = lax.rsqrt(deg)
        dinv_ref[...] = dinv
        t_ref[...] = _dot(x_ref[...], w_ref[...]) * dinv

    return pl.pallas_call(
        body,
        grid=(N // RB,),
        in_specs=[
            pl.BlockSpec((RB, D), lambda i: (i, 0)),
            pl.BlockSpec((D, D), lambda i: (0, 0)),
            pl.BlockSpec((1, RB, DEGW), lambda i: (0, i, 0)),
            pl.BlockSpec((1, RB, DEGW), lambda i: (1, i, 0)),
        ],
        out_specs=[
            pl.BlockSpec((RB, D), lambda i: (i, 0)),
            pl.BlockSpec((RB, 1), lambda i: (i, 0)),
        ],
        out_shape=[
            jax.ShapeDtypeStruct((N, D), jnp.float32),
            jax.ShapeDtypeStruct((N, 1), jnp.float32),
        ],
    )(x, w1, cnt, cnt)


def _tc_layer_next(part, t_prev, dinv, b_prev, w_next):
    def body(pa_ref, pb_ref, tp_ref, dinv_ref, b_ref, w_ref, t_ref):
        dinv = dinv_ref[...]
        agg = pa_ref[0] + pb_ref[0] - tp_ref[...]
        h = _elu(dinv * agg + b_ref[...])
        t_ref[...] = _dot(h, w_ref[...]) * dinv

    return pl.pallas_call(
        body,
        grid=(N // RB,),
        in_specs=[
            pl.BlockSpec((1, RB, D), lambda i: (0, i, 0)),
            pl.BlockSpec((1, RB, D), lambda i: (1, i, 0)),
            pl.BlockSpec((RB, D), lambda i: (i, 0)),
            pl.BlockSpec((RB, 1), lambda i: (i, 0)),
            pl.BlockSpec((1, D), lambda i: (0, 0)),
            pl.BlockSpec((D, D), lambda i: (0, 0)),
        ],
        out_specs=pl.BlockSpec((RB, D), lambda i: (i, 0)),
        out_shape=jax.ShapeDtypeStruct((N, D), jnp.float32),
    )(part, part, t_prev, dinv, b_prev, w_next)


def _tc_offsets(batch_col):
    def body(b_ref, o_ref):
        hist = jnp.zeros((1, G), jnp.float32)
        for i in range(N // RB):
            vals = b_ref[i * RB:(i + 1) * RB, :]
            eq = (vals == lax.broadcasted_iota(jnp.int32, (RB, G), 1))
            hist = hist + jnp.sum(eq.astype(jnp.float32), axis=0, keepdims=True)
        row = lax.broadcasted_iota(jnp.int32, (G, G), 0)
        col = lax.broadcasted_iota(jnp.int32, (G, G), 1)
        strict_lower = (row < col).astype(jnp.float32)
        starts = _dot(hist, strict_lower)
        o_ref[...] = starts.astype(jnp.int32)

    return pl.pallas_call(
        body,
        in_specs=[pl.BlockSpec((N, 1), lambda: (0, 0))],
        out_specs=pl.BlockSpec((1, G), lambda: (0, 0)),
        out_shape=jax.ShapeDtypeStruct((1, G), jnp.int32),
    )(batch_col)


def _tc_pool_head(part3, t3, dinv, b3, wo, bo, starts):
    def body(starts_ref, pa_ref, pb_ref, tp_ref, dinv_ref, b_ref, wo_ref,
             bo_ref, o_ref, pooled):
        g = pl.program_id(0)

        @pl.when(g < G)
        def _():
            s = starts_ref[g]
            e = jnp.where(g == G - 1, N, starts_ref[jnp.minimum(g + 1, G - 1)])
            c0 = (s // 8) * 8
            nch = (e - c0 + 7) // 8

            def chunk(i, acc):
                r = c0 + 8 * i
                agg = (pa_ref[0, pl.ds(r, 8), :] + pb_ref[0, pl.ds(r, 8), :]
                       - tp_ref[pl.ds(r, 8), :])
                dinv = dinv_ref[pl.ds(r, 8), :]
                h = _elu(dinv * agg + b_ref[...])
                rid = r + lax.broadcasted_iota(jnp.int32, (8, D), 0)
                ok = jnp.logical_and(rid >= s, rid < e)
                return jnp.maximum(acc, jnp.where(ok, h, -1e30))

            acc = lax.fori_loop(0, nch, chunk,
                                jnp.full((8, D), -1e30, jnp.float32))
            pooled[pl.ds(g, 1), :] = jnp.max(acc, axis=0, keepdims=True)

        @pl.when(g == G)
        def _():
            p = pooled[...]
            p = jnp.where(p < -1e29, 0.0, p)
            logits = _dot(p, wo_ref[...]) + bo_ref[...]
            m = jnp.max(logits, axis=1, keepdims=True)
            ex = jnp.exp(logits - m)
            o_ref[...] = ex / jnp.sum(ex, axis=1, keepdims=True)

    grid_spec = pltpu.PrefetchScalarGridSpec(
        num_scalar_prefetch=1,
        grid=(G + 1,),
        in_specs=[
            pl.BlockSpec((1, N, D), lambda g, s_ref: (0, 0, 0)),
            pl.BlockSpec((1, N, D), lambda g, s_ref: (1, 0, 0)),
            pl.BlockSpec((N, D), lambda g, s_ref: (0, 0)),
            pl.BlockSpec((N, 1), lambda g, s_ref: (0, 0)),
            pl.BlockSpec((1, D), lambda g, s_ref: (0, 0)),
            pl.BlockSpec((D, NCLS), lambda g, s_ref: (0, 0)),
            pl.BlockSpec((1, NCLS), lambda g, s_ref: (0, 0)),
        ],
        out_specs=pl.BlockSpec((G, NCLS), lambda g, s_ref: (0, 0)),
        scratch_shapes=[pltpu.VMEM((G, D), jnp.float32)],
    )
    return pl.pallas_call(
        body,
        grid_spec=grid_spec,
        out_shape=jax.ShapeDtypeStruct((G, NCLS), jnp.float32),
    )(starts, part3, part3, t3, dinv, b3, wo, bo)


def kernel(x, edge_index, batch, W1, b1, W2, b2, W3, b3, Wo, bo):
    src_w = edge_index[0].astype(jnp.int32).reshape(NW, NCHUNK, CHUNK)
    dst_w = edge_index[1].astype(jnp.int32).reshape(NW, NCHUNK, CHUNK)
    batch_col = batch.astype(jnp.int32).reshape(N, 1)
    ones_hbm = jnp.ones((RPT, DEGW), jnp.float32)

    cnt = _sc_degree(ones_hbm, dst_w)
    t1, dinv = _tc_layer1(x, W1, cnt)
    p1 = _sc_aggregate(t1, src_w, dst_w)
    t2 = _tc_layer_next(p1, t1, dinv, b1.reshape(1, D), W2)
    p2 = _sc_aggregate(t2, src_w, dst_w)
    t3 = _tc_layer_next(p2, t2, dinv, b2.reshape(1, D), W3)
    p3 = _sc_aggregate(t3, src_w, dst_w)
    starts = _tc_offsets(batch_col)
    return _tc_pool_head(p3, t3, dinv, b3.reshape(1, D), Wo,
                         bo.reshape(1, NCLS), starts.reshape(G))
